# Initial kernel scaffold; baseline (speedup 1.0000x reference)
import jax
import jax.numpy as jnp
from jax.experimental import pallas as pl


def kernel(x, edge_index_0, edge_index_1, W1_l, b1, W1_r, W2_l, b2, W2_r):
    raise NotImplementedError("write your pallas kernel here")


# speedup vs baseline: 5.3883x; 5.3883x over previous
import functools

import jax
import jax.numpy as jnp
from jax import lax
from jax.experimental import pallas as pl
from jax.experimental.pallas import tpu as pltpu
from jax.experimental.pallas import tpu_sc as plsc

N0 = 10000
N1 = 4096
N2 = 1024
E0 = 320000
E1 = 65536
D = 128
H = 128

NC = 2
NS = 16
NW = NC * NS
CH = 128


def _make_seg_sum(e_pad: int, n_acc: int):
    per_w = e_pad // NW
    nb = per_w // CH
    assert per_w % CH == 0 and e_pad % NW == 0 and n_acc % 16 == 0

    mesh = plsc.VectorSubcoreMesh(core_axis_name="c", subcore_axis_name="s")

    @functools.partial(
        pl.kernel,
        out_type=(
            jax.ShapeDtypeStruct((NC, n_acc, D), jnp.float32),
            jax.ShapeDtypeStruct((NW, n_acc), jnp.float32),
        ),
        mesh=mesh,
        compiler_params=pltpu.CompilerParams(needs_layout_passes=False),
        scratch_types=[
            pltpu.VMEM((CH,), jnp.int32),
            pltpu.VMEM((CH,), jnp.int32),
            pltpu.VMEM((CH, D), jnp.float32),
            pltpu.VMEM((n_acc,), jnp.float32),
            pltpu.VMEM_SHARED((n_acc, D), jnp.float32),
            pltpu.SemaphoreType.DMA,
        ],
    )
    def seg_kernel(x_hbm, src_hbm, dst_hbm, zagg_hbm, zhist_hbm,
                   agg_out, cnt_out,
                   src_v, dst_v, rows_v, hist_v, agg_sh, sem):
        c = lax.axis_index("c")
        s = lax.axis_index("s")
        wid = s * NC + c

        @pl.when(s == 0)
        def _zero():
            pltpu.sync_copy(zagg_hbm, agg_sh)

        pltpu.sync_copy(zhist_hbm, hist_v)
        plsc.subcore_barrier()

        base = wid * per_w
        ones = jnp.ones((16,), jnp.float32)

        def step(k, carry):
            off = base + k * CH
            pltpu.sync_copy(src_hbm.at[pl.ds(off, CH)], src_v)
            pltpu.sync_copy(dst_hbm.at[pl.ds(off, CH)], dst_v)
            pltpu.async_copy(x_hbm.at[src_v], rows_v, sem).wait()
            pltpu.sync_copy(rows_v, agg_sh.at[dst_v], add=True)
            for j in range(CH // 16):
                dv = dst_v[pl.ds(j * 16, 16)]
                plsc.addupdate_scatter(hist_v, [dv], ones)
            return carry

        lax.fori_loop(0, nb, step, 0)
        pltpu.sync_copy(hist_v, cnt_out.at[wid])
        plsc.subcore_barrier()

        @pl.when(s == 0)
        def _writeout():
            pltpu.sync_copy(agg_sh, agg_out.at[c])

    return seg_kernel


def _combine(agg_parts, cnt_t, x_dst, w_l, b, w_r, leaky: bool):
    n = x_dst.shape[0]

    def body(ap, cp, xd, wl, bb, wr, o):
        agg = ap[0, :n, :] + ap[1, :n, :]
        cnt = jnp.sum(cp[...], axis=1, keepdims=True)
        mean = agg / jnp.maximum(cnt, 1.0)
        r = (lax.dot_general(mean, wl[...], (((1,), (1,)), ((), ())),
                             preferred_element_type=jnp.float32)
             + bb[...]
             + lax.dot_general(xd[...], wr[...], (((1,), (1,)), ((), ())),
                               preferred_element_type=jnp.float32))
        if leaky:
            r = jnp.where(r >= 0, r, 0.01 * r)
        o[...] = r

    return pl.pallas_call(
        body,
        out_shape=jax.ShapeDtypeStruct((n, H), jnp.float32),
    )(agg_parts, cnt_t, x_dst, w_l, b.reshape(1, H), w_r)


E0_PAD = ((E0 + NW * CH - 1) // (NW * CH)) * (NW * CH)
E1_PAD = ((E1 + NW * CH - 1) // (NW * CH)) * (NW * CH)
NACC0 = 33 * 128
NACC1 = 9 * 128

_seg0 = _make_seg_sum(E0_PAD, NACC0)
_seg1 = _make_seg_sum(E1_PAD, NACC1)


def _pad_edges(ei, e_pad, n_dst):
    src, dst = ei[0], ei[1]
    pad = e_pad - src.shape[0]
    if pad:
        src = jnp.concatenate([src, jnp.zeros((pad,), src.dtype)])
        dst = jnp.concatenate([dst, jnp.full((pad,), n_dst, dst.dtype)])
    return src, dst


def kernel(x, edge_index_0, edge_index_1, W1_l, b1, W1_r, W2_l, b2, W2_r):
    src0, dst0 = _pad_edges(edge_index_0, E0_PAD, N1)
    src1, dst1 = _pad_edges(edge_index_1, E1_PAD, N2)

    zagg0 = jnp.zeros((NACC0, D), jnp.float32)
    zh0 = jnp.zeros((NACC0,), jnp.float32)
    zagg1 = jnp.zeros((NACC1, D), jnp.float32)
    zh1 = jnp.zeros((NACC1,), jnp.float32)

    agg0, cnt0 = _seg0(x, src0, dst0, zagg0, zh0)
    h = _combine(agg0, cnt0[:, :N1].T, x[:N1], W1_l, b1, W1_r, leaky=True)
    agg1, cnt1 = _seg1(h, src1, dst1, zagg1, zh1)
    out = _combine(agg1, cnt1[:, :N2].T, h[:N2], W2_l, b2, W2_r, leaky=False)
    return out
